# baseline (speedup 1.0000x reference)
import functools

import jax
import jax.numpy as jnp
import numpy as np
from jax.experimental import pallas as pl
from jax.experimental.pallas import tpu as pltpu

_ROT = ((13, 15, 26, 6), (17, 29, 16, 24))
_TINY = np.float32(np.finfo(np.float32).tiny)
_SPAN = np.float32(np.float32(1.0) - _TINY)
_BIG_IDX = np.int32(2**30)


def _rotl(x, d):
    return (x << np.uint32(d)) | (x >> np.uint32(32 - d))


def _threefry2x32(x0, x1):
    ks = (np.uint32(0), np.uint32(42),
          np.uint32(np.uint32(0) ^ np.uint32(42) ^ np.uint32(0x1BD11BDA)))
    x0 = x0 + ks[0]
    x1 = x1 + ks[1]
    for i in range(5):
        for r in _ROT[i % 2]:
            x0 = x0 + x1
            x1 = _rotl(x1, r)
            x1 = x1 ^ x0
        x0 = x0 + ks[(i + 1) % 3]
        x1 = x1 + ks[(i + 2) % 3] + np.uint32(i + 1)
    return x0 ^ x1


def _sample_kernel(logits_ref, out_ref, rv_ref, ri_ref, *, ncols, width,
                   rows_per_blk, nblk):
    r = pl.program_id(0)
    k = pl.program_id(1)

    @pl.when(k == 0)
    def _init():
        rv_ref[...] = jnp.full_like(rv_ref, -jnp.inf)
        ri_ref[...] = jnp.full_like(ri_ref, _BIG_IDX)

    shape = (rows_per_blk, width)
    row = (jax.lax.broadcasted_iota(jnp.int32, shape, 0)
           + r * rows_per_blk)
    col = jax.lax.broadcasted_iota(jnp.int32, shape, 1) + k * width
    lin = row * ncols + col

    bits = _threefry2x32(jnp.zeros(shape, jnp.uint32), lin.astype(jnp.uint32))
    fb = (bits >> np.uint32(9)) | np.uint32(0x3F800000)
    f = jax.lax.bitcast_convert_type(fb, jnp.float32) - np.float32(1.0)
    u = jnp.maximum(_TINY, f * _SPAN + _TINY)
    g = -jnp.log(-jnp.log(u))

    val = g + logits_ref[...]
    val = jnp.where(col < ncols, val, -jnp.inf)

    upd = val > rv_ref[...]
    rv_ref[...] = jnp.where(upd, val, rv_ref[...])
    ri_ref[...] = jnp.where(upd, col, ri_ref[...])

    @pl.when(k == nblk - 1)
    def _finish():
        rv = rv_ref[...]
        ri = ri_ref[...]
        m = jnp.max(rv, axis=1, keepdims=True)
        cand = jnp.where(rv == m, ri, _BIG_IDX)
        out_ref[...] = jnp.min(cand, axis=1, keepdims=True)


@jax.jit
def kernel(logits):
    nrows, ncols = logits.shape
    rows_per_blk = nrows // 2
    width = 4096
    nblk = pl.cdiv(ncols, width)

    out = pl.pallas_call(
        functools.partial(_sample_kernel, ncols=ncols, width=width,
                          rows_per_blk=rows_per_blk, nblk=nblk),
        grid=(2, nblk),
        in_specs=[pl.BlockSpec((rows_per_blk, width), lambda r, k: (r, k))],
        out_specs=pl.BlockSpec((rows_per_blk, 1), lambda r, k: (r, 0)),
        out_shape=jax.ShapeDtypeStruct((nrows, 1), jnp.int32),
        scratch_shapes=[
            pltpu.VMEM((rows_per_blk, width), jnp.float32),
            pltpu.VMEM((rows_per_blk, width), jnp.int32),
        ],
        compiler_params=pltpu.CompilerParams(
            dimension_semantics=("parallel", "arbitrary"),
        ),
    )(logits)
    return out.reshape(nrows).astype(jnp.int64)
